# baseline (speedup 1.0000x reference)
import jax
import jax.numpy as jnp
from jax.experimental import pallas as pl

_B = 128
_L = 2048
_H = 64
_V = 64
_K = 4
_CHUNK = 128


def _fused_kernel(seq_ref, embed_ref, W1_ref, b1_ref, W2_ref, b2_ref,
                  gamma_ref, beta_ref, Wg1_ref, bg1_ref, Wg2_ref, bg2_ref,
                  Wq_ref, bq_ref, Wout_ref, bout_ref, out_ref):
    f32 = jnp.float32
    hi = jax.lax.Precision.HIGHEST

    emb = embed_ref[...]
    ff1 = jnp.maximum(
        jnp.dot(emb, W1_ref[...], preferred_element_type=f32,
                precision=hi) + b1_ref[...], 0.0)
    ff = jnp.dot(ff1, W2_ref[...], preferred_element_type=f32,
                 precision=hi) + b2_ref[...]
    x = emb + ff
    mean = jnp.mean(x, axis=1, keepdims=True)
    var = jnp.mean((x - mean) ** 2, axis=1, keepdims=True)
    h = (x - mean) / jnp.sqrt(var + 1e-5) * gamma_ref[...] + beta_ref[...]

    g1 = jnp.maximum(
        jnp.dot(h, Wg1_ref[...], preferred_element_type=f32,
                precision=hi) + bg1_ref[...], 0.0)
    gl = jnp.dot(g1, Wg2_ref[...], preferred_element_type=f32,
                 precision=hi) + bg2_ref[...]

    iota_col = jax.lax.broadcasted_iota(jnp.int32, (_V, 1), 0)
    iota_row = jax.lax.broadcasted_iota(jnp.int32, (1, _V), 1)

    g_col = gl
    g_bcast = gl + jnp.zeros((1, _V), f32)
    g_row = jnp.sum(jnp.where(iota_col == iota_row, g_bcast, 0.0),
                    axis=0, keepdims=True)

    bigger = (g_col > g_row) | ((g_col == g_row) & (iota_col < iota_row))
    r_row = jnp.sum(bigger.astype(f32), axis=0, keepdims=True)
    bigger2 = (g_row > g_col) | ((g_row == g_col) & (iota_row < iota_col))
    r_col = jnp.sum(bigger2.astype(f32), axis=1, keepdims=True)

    perm = (iota_col.astype(f32) == r_row).astype(f32)
    perm_t = (r_col == iota_row.astype(f32)).astype(f32)

    h_sorted = jnp.dot(perm, h, preferred_element_type=f32, precision=hi)

    iota_v3 = jax.lax.broadcasted_iota(jnp.int32, (1, _V, 1), 1)
    def hist_step(i, acc):
        tok = seq_ref[:, pl.ds(i * _CHUNK, _CHUNK)]
        oh = (tok[:, None, :] == iota_v3).astype(f32)
        return acc + jnp.sum(oh, axis=2)

    counts = jax.lax.fori_loop(0, _L // _CHUNK, hist_step,
                               jnp.zeros((_B, _V), f32))

    counts_sorted = jnp.dot(counts, perm_t, preferred_element_type=f32, precision=hi)
    lower_tri = (iota_col <= iota_row).astype(f32)
    cum = jnp.dot(counts_sorted, lower_tri, preferred_element_type=f32, precision=hi)
    cum_excl = cum - counts_sorted

    tq = seq_ref[:, _L - 1:_L]
    q_onehot = (tq == iota_row).astype(f32)
    query_h = jnp.dot(q_onehot, h, preferred_element_type=f32, precision=hi)
    q = jnp.dot(query_h, Wq_ref[...], preferred_element_type=f32, precision=hi) + bq_ref[...]

    slots = []
    scores = []
    for j in range(_K):
        sel = ((cum_excl <= j) & (cum > j)).astype(f32)
        slot = jnp.dot(sel, h_sorted, preferred_element_type=f32, precision=hi)
        slots.append(slot)
        scores.append(jnp.sum(slot * q, axis=1, keepdims=True) * 0.125)

    smax = jnp.maximum(jnp.maximum(scores[0], scores[1]),
                       jnp.maximum(scores[2], scores[3]))
    exps = [jnp.exp(s - smax) for s in scores]
    denom = exps[0] + exps[1] + exps[2] + exps[3]
    pooled = sum(e * s for e, s in zip(exps, slots)) / denom

    out_ref[...] = (jnp.dot(pooled, Wout_ref[...], preferred_element_type=f32, precision=hi)
                    + bout_ref[...])


def kernel(seq, embed, W1, b1, W2, b2, gamma, beta, Wg1, bg1, Wg2, bg2,
           Wq, bq, Wout, bout):
    row = lambda a: a.reshape(1, -1)
    return pl.pallas_call(
        _fused_kernel,
        out_shape=jax.ShapeDtypeStruct((_B, _H), jnp.float32),
    )(seq.astype(jnp.int32), embed, W1, row(b1), W2, row(b2), row(gamma),
      row(beta), Wg1, row(bg1), Wg2, row(bg2), Wq, row(bq), Wout, row(bout))


# speedup vs baseline: 1.3032x; 1.3032x over previous
import jax
import jax.numpy as jnp
from jax.experimental import pallas as pl

_B = 128
_L = 2048
_H = 64
_V = 64
_K = 4
_CHUNK = 128


def _fused_kernel(seq_ref, embed_ref, W1_ref, b1_ref, W2_ref, b2_ref,
                  gamma_ref, beta_ref, Wg1_ref, bg1_ref, Wg2_ref, bg2_ref,
                  Wq_ref, bq_ref, Wout_ref, bout_ref, out_ref):
    f32 = jnp.float32
    hi = jax.lax.Precision.HIGHEST

    emb = embed_ref[...]
    ff1 = jnp.maximum(
        jnp.dot(emb, W1_ref[...], preferred_element_type=f32,
                precision=hi) + b1_ref[...], 0.0)
    ff = jnp.dot(ff1, W2_ref[...], preferred_element_type=f32,
                 precision=hi) + b2_ref[...]
    x = emb + ff
    mean = jnp.mean(x, axis=1, keepdims=True)
    var = jnp.mean((x - mean) ** 2, axis=1, keepdims=True)
    h = (x - mean) / jnp.sqrt(var + 1e-5) * gamma_ref[...] + beta_ref[...]

    g1 = jnp.maximum(
        jnp.dot(h, Wg1_ref[...], preferred_element_type=f32,
                precision=hi) + bg1_ref[...], 0.0)
    gl = jnp.dot(g1, Wg2_ref[...], preferred_element_type=f32,
                 precision=hi) + bg2_ref[...]

    iota_col = jax.lax.broadcasted_iota(jnp.int32, (_V, 1), 0)
    iota_row = jax.lax.broadcasted_iota(jnp.int32, (1, _V), 1)

    g_col = gl
    g_bcast = gl + jnp.zeros((1, _V), f32)
    g_row = jnp.sum(jnp.where(iota_col == iota_row, g_bcast, 0.0),
                    axis=0, keepdims=True)

    bigger = (g_col > g_row) | ((g_col == g_row) & (iota_col < iota_row))
    r_row = jnp.sum(bigger.astype(f32), axis=0, keepdims=True)
    bigger2 = (g_row > g_col) | ((g_row == g_col) & (iota_row < iota_col))
    r_col = jnp.sum(bigger2.astype(f32), axis=1, keepdims=True)

    perm = (iota_col.astype(f32) == r_row).astype(f32)
    perm_t = (r_col == iota_row.astype(f32)).astype(f32)

    h_sorted = jnp.dot(perm, h, preferred_element_type=f32, precision=hi)

    iota_v8 = jax.lax.broadcasted_iota(jnp.int32, (1, 8, 1), 1)
    def hist_step(i, acc):
        tok = seq_ref[:, pl.ds(i * _CHUNK, _CHUNK)]
        tok3 = tok[:, None, :]
        parts = []
        for vt in range(_V // 8):
            m = (tok3 == iota_v8 + vt * 8).astype(f32)
            parts.append(jnp.sum(m, axis=2))
        return acc + jnp.concatenate(parts, axis=1)

    counts = jax.lax.fori_loop(0, _L // _CHUNK, hist_step,
                               jnp.zeros((_B, _V), f32))

    counts_sorted = jnp.dot(counts, perm_t, preferred_element_type=f32, precision=hi)
    lower_tri = (iota_col <= iota_row).astype(f32)
    cum = jnp.dot(counts_sorted, lower_tri, preferred_element_type=f32, precision=hi)
    cum_excl = cum - counts_sorted

    tq = seq_ref[:, _L - 1:_L]
    q_onehot = (tq == iota_row).astype(f32)
    query_h = jnp.dot(q_onehot, h, preferred_element_type=f32, precision=hi)
    q = jnp.dot(query_h, Wq_ref[...], preferred_element_type=f32, precision=hi) + bq_ref[...]

    slots = []
    scores = []
    for j in range(_K):
        sel = ((cum_excl <= j) & (cum > j)).astype(f32)
        slot = jnp.dot(sel, h_sorted, preferred_element_type=f32, precision=hi)
        slots.append(slot)
        scores.append(jnp.sum(slot * q, axis=1, keepdims=True) * 0.125)

    smax = jnp.maximum(jnp.maximum(scores[0], scores[1]),
                       jnp.maximum(scores[2], scores[3]))
    exps = [jnp.exp(s - smax) for s in scores]
    denom = exps[0] + exps[1] + exps[2] + exps[3]
    pooled = sum(e * s for e, s in zip(exps, slots)) / denom

    out_ref[...] = (jnp.dot(pooled, Wout_ref[...], preferred_element_type=f32, precision=hi)
                    + bout_ref[...])


def kernel(seq, embed, W1, b1, W2, b2, gamma, beta, Wg1, bg1, Wg2, bg2,
           Wq, bq, Wout, bout):
    row = lambda a: a.reshape(1, -1)
    return pl.pallas_call(
        _fused_kernel,
        out_shape=jax.ShapeDtypeStruct((_B, _H), jnp.float32),
    )(seq.astype(jnp.int32), embed, W1, row(b1), W2, row(b2), row(gamma),
      row(beta), Wg1, row(bg1), Wg2, row(bg2), Wq, row(bq), Wout, row(bout))


# speedup vs baseline: 1.9195x; 1.4729x over previous
import jax
import jax.numpy as jnp
from jax.experimental import pallas as pl

_B = 128
_L = 2048
_H = 64
_V = 64
_K = 4
_CHUNK = 256


def _fused_kernel(seq_ref, embed_ref, W1_ref, b1_ref, W2_ref, b2_ref,
                  gamma_ref, beta_ref, Wg1_ref, bg1_ref, Wg2_ref, bg2_ref,
                  Wq_ref, bq_ref, Wout_ref, bout_ref, out_ref):
    f32 = jnp.float32
    hi = jax.lax.Precision.HIGHEST

    emb = embed_ref[...]
    ff1 = jnp.maximum(
        jnp.dot(emb, W1_ref[...], preferred_element_type=f32,
                precision=hi) + b1_ref[...], 0.0)
    ff = jnp.dot(ff1, W2_ref[...], preferred_element_type=f32,
                 precision=hi) + b2_ref[...]
    x = emb + ff
    mean = jnp.mean(x, axis=1, keepdims=True)
    var = jnp.mean((x - mean) ** 2, axis=1, keepdims=True)
    h = (x - mean) / jnp.sqrt(var + 1e-5) * gamma_ref[...] + beta_ref[...]

    g1 = jnp.maximum(
        jnp.dot(h, Wg1_ref[...], preferred_element_type=f32,
                precision=hi) + bg1_ref[...], 0.0)
    gl = jnp.dot(g1, Wg2_ref[...], preferred_element_type=f32,
                 precision=hi) + bg2_ref[...]

    iota_col = jax.lax.broadcasted_iota(jnp.int32, (_V, 1), 0)
    iota_row = jax.lax.broadcasted_iota(jnp.int32, (1, _V), 1)

    g_col = gl
    g_bcast = gl + jnp.zeros((1, _V), f32)
    g_row = jnp.sum(jnp.where(iota_col == iota_row, g_bcast, 0.0),
                    axis=0, keepdims=True)

    bigger = (g_col > g_row) | ((g_col == g_row) & (iota_col < iota_row))
    r_row = jnp.sum(bigger.astype(f32), axis=0, keepdims=True)
    bigger2 = (g_row > g_col) | ((g_row == g_col) & (iota_row < iota_col))
    r_col = jnp.sum(bigger2.astype(f32), axis=1, keepdims=True)

    perm = (iota_col.astype(f32) == r_row).astype(f32)
    perm_t = (r_col == iota_row.astype(f32)).astype(f32)

    h_sorted = jnp.dot(perm, h, preferred_element_type=f32, precision=hi)

    iota_v8 = jax.lax.broadcasted_iota(jnp.int32, (1, 8, 1), 1)
    def hist_step(i, acc):
        tok = seq_ref[:, pl.ds(i * _CHUNK, _CHUNK)]
        tok3 = tok[:, None, :]
        parts = []
        for vt in range(_V // 8):
            m = (tok3 == iota_v8 + vt * 8).astype(f32)
            parts.append(jnp.sum(m, axis=2))
        return acc + jnp.concatenate(parts, axis=1)

    counts = jax.lax.fori_loop(0, _L // _CHUNK, hist_step,
                               jnp.zeros((_B, _V), f32))

    counts_sorted = jnp.dot(counts, perm_t, preferred_element_type=f32, precision=hi)
    lower_tri = (iota_col <= iota_row).astype(f32)
    cum = jnp.dot(counts_sorted, lower_tri, preferred_element_type=f32, precision=hi)
    cum_excl = cum - counts_sorted

    tq = seq_ref[:, _L - 1:_L]
    q_onehot = (tq == iota_row).astype(f32)
    query_h = jnp.dot(q_onehot, h, preferred_element_type=f32, precision=hi)
    q = jnp.dot(query_h, Wq_ref[...], preferred_element_type=f32, precision=hi) + bq_ref[...]

    slots = []
    scores = []
    for j in range(_K):
        sel = ((cum_excl <= j) & (cum > j)).astype(f32)
        slot = jnp.dot(sel, h_sorted, preferred_element_type=f32, precision=hi)
        slots.append(slot)
        scores.append(jnp.sum(slot * q, axis=1, keepdims=True) * 0.125)

    smax = jnp.maximum(jnp.maximum(scores[0], scores[1]),
                       jnp.maximum(scores[2], scores[3]))
    exps = [jnp.exp(s - smax) for s in scores]
    denom = exps[0] + exps[1] + exps[2] + exps[3]
    pooled = sum(e * s for e, s in zip(exps, slots)) / denom

    out_ref[...] = (jnp.dot(pooled, Wout_ref[...], preferred_element_type=f32, precision=hi)
                    + bout_ref[...])


def kernel(seq, embed, W1, b1, W2, b2, gamma, beta, Wg1, bg1, Wg2, bg2,
           Wq, bq, Wout, bout):
    row = lambda a: a.reshape(1, -1)
    return pl.pallas_call(
        _fused_kernel,
        out_shape=jax.ShapeDtypeStruct((_B, _H), jnp.float32),
    )(seq.astype(jnp.int32), embed, W1, row(b1), W2, row(b2), row(gamma),
      row(beta), Wg1, row(bg1), Wg2, row(bg2), Wq, row(bq), Wout, row(bout))


# speedup vs baseline: 2.1454x; 1.1177x over previous
import jax
import jax.numpy as jnp
from jax.experimental import pallas as pl

_B = 128
_L = 2048
_H = 64
_V = 64
_K = 4
_CHUNK = 512


def _fused_kernel(seq_ref, embed_ref, W1_ref, b1_ref, W2_ref, b2_ref,
                  gamma_ref, beta_ref, Wg1_ref, bg1_ref, Wg2_ref, bg2_ref,
                  Wq_ref, bq_ref, Wout_ref, bout_ref, out_ref):
    f32 = jnp.float32
    hi = jax.lax.Precision.HIGHEST

    emb = embed_ref[...]
    ff1 = jnp.maximum(
        jnp.dot(emb, W1_ref[...], preferred_element_type=f32,
                precision=hi) + b1_ref[...], 0.0)
    ff = jnp.dot(ff1, W2_ref[...], preferred_element_type=f32,
                 precision=hi) + b2_ref[...]
    x = emb + ff
    mean = jnp.mean(x, axis=1, keepdims=True)
    var = jnp.mean((x - mean) ** 2, axis=1, keepdims=True)
    h = (x - mean) / jnp.sqrt(var + 1e-5) * gamma_ref[...] + beta_ref[...]

    g1 = jnp.maximum(
        jnp.dot(h, Wg1_ref[...], preferred_element_type=f32,
                precision=hi) + bg1_ref[...], 0.0)
    gl = jnp.dot(g1, Wg2_ref[...], preferred_element_type=f32,
                 precision=hi) + bg2_ref[...]

    iota_col = jax.lax.broadcasted_iota(jnp.int32, (_V, 1), 0)
    iota_row = jax.lax.broadcasted_iota(jnp.int32, (1, _V), 1)

    g_col = gl
    g_bcast = gl + jnp.zeros((1, _V), f32)
    g_row = jnp.sum(jnp.where(iota_col == iota_row, g_bcast, 0.0),
                    axis=0, keepdims=True)

    bigger = (g_col > g_row) | ((g_col == g_row) & (iota_col < iota_row))
    r_row = jnp.sum(bigger.astype(f32), axis=0, keepdims=True)
    bigger2 = (g_row > g_col) | ((g_row == g_col) & (iota_row < iota_col))
    r_col = jnp.sum(bigger2.astype(f32), axis=1, keepdims=True)

    perm = (iota_col.astype(f32) == r_row).astype(f32)
    perm_t = (r_col == iota_row.astype(f32)).astype(f32)

    h_sorted = jnp.dot(perm, h, preferred_element_type=f32, precision=hi)

    iota_v8 = jax.lax.broadcasted_iota(jnp.int32, (1, 8, 1), 1)
    def hist_step(i, acc):
        tok = seq_ref[:, pl.ds(i * _CHUNK, _CHUNK)]
        tok3 = tok[:, None, :]
        parts = []
        for vt in range(_V // 8):
            m = (tok3 == iota_v8 + vt * 8).astype(f32)
            parts.append(jnp.sum(m, axis=2))
        return acc + jnp.concatenate(parts, axis=1)

    counts = jax.lax.fori_loop(0, _L // _CHUNK, hist_step,
                               jnp.zeros((_B, _V), f32))

    counts_sorted = jnp.dot(counts, perm_t, preferred_element_type=f32, precision=hi)
    lower_tri = (iota_col <= iota_row).astype(f32)
    cum = jnp.dot(counts_sorted, lower_tri, preferred_element_type=f32, precision=hi)
    cum_excl = cum - counts_sorted

    tq = seq_ref[:, _L - 1:_L]
    q_onehot = (tq == iota_row).astype(f32)
    query_h = jnp.dot(q_onehot, h, preferred_element_type=f32, precision=hi)
    q = jnp.dot(query_h, Wq_ref[...], preferred_element_type=f32, precision=hi) + bq_ref[...]

    slots = []
    scores = []
    for j in range(_K):
        sel = ((cum_excl <= j) & (cum > j)).astype(f32)
        slot = jnp.dot(sel, h_sorted, preferred_element_type=f32, precision=hi)
        slots.append(slot)
        scores.append(jnp.sum(slot * q, axis=1, keepdims=True) * 0.125)

    smax = jnp.maximum(jnp.maximum(scores[0], scores[1]),
                       jnp.maximum(scores[2], scores[3]))
    exps = [jnp.exp(s - smax) for s in scores]
    denom = exps[0] + exps[1] + exps[2] + exps[3]
    pooled = sum(e * s for e, s in zip(exps, slots)) / denom

    out_ref[...] = (jnp.dot(pooled, Wout_ref[...], preferred_element_type=f32, precision=hi)
                    + bout_ref[...])


def kernel(seq, embed, W1, b1, W2, b2, gamma, beta, Wg1, bg1, Wg2, bg2,
           Wq, bq, Wout, bout):
    row = lambda a: a.reshape(1, -1)
    return pl.pallas_call(
        _fused_kernel,
        out_shape=jax.ShapeDtypeStruct((_B, _H), jnp.float32),
    )(seq.astype(jnp.int32), embed, W1, row(b1), W2, row(b2), row(gamma),
      row(beta), Wg1, row(bg1), Wg2, row(bg2), Wq, row(bq), Wout, row(bout))
